# baseline (speedup 1.0000x reference)
import math
import functools

import jax
import jax.numpy as jnp
from jax import lax
from jax.experimental import pallas as pl

_NBINS = 10
_POOL = 8
_H = 512
_W = 512


def _hog_body(x_ref, o_ref):
    img = x_ref[0].astype(jnp.bfloat16).astype(jnp.float32)
    zrow = jnp.zeros((1, _W), dtype=jnp.float32)
    zcol = jnp.zeros((_H, 1), dtype=jnp.float32)
    x_up = jnp.concatenate([img[1:, :], zrow], axis=0)
    x_dn = jnp.concatenate([zrow, img[:-1, :]], axis=0)

    t1 = x_dn + 2.0 * img + x_up
    t2 = x_dn - x_up

    t1_l = jnp.concatenate([zcol, t1[:, :-1]], axis=1)
    t1_r = jnp.concatenate([t1[:, 1:], zcol], axis=1)
    t2_l = jnp.concatenate([zcol, t2[:, :-1]], axis=1)
    t2_r = jnp.concatenate([t2[:, 1:], zcol], axis=1)

    gx = t1_l - t1_r
    gy = t2_l + 2.0 * t2 + t2_r

    mag = jnp.sqrt(gx * gx + gy * gy)

    s = jnp.where(gx > 0.0, 1.0, jnp.where(gx < 0.0, -1.0,
                  jnp.where(gy < 0.0, -1.0, 1.0))).astype(jnp.float32)
    u = s * gx
    v = s * gy


    rows = lax.broadcasted_iota(jnp.int32, (_H, _H // _POOL), 0)
    cols = lax.broadcasted_iota(jnp.int32, (_H, _H // _POOL), 1)
    pmat = jnp.where(rows // _POOL == cols, 1.0 / _POOL, 0.0).astype(jnp.float32)

    pooled = [None] * (_NBINS + 1)
    for j in range(_NBINS):
        if j == 0:
            mj = mag
        else:
            cj = (u * jnp.float32(math.cos(j * math.pi / _NBINS))
                  - v * jnp.float32(math.sin(j * math.pi / _NBINS))) >= 0.0
            mj = jnp.where(cj, mag, 0.0)
        ph = lax.dot_general(pmat, mj, (((0,), (0,)), ((), ())),
                             preferred_element_type=jnp.float32,
                             precision=lax.Precision.HIGHEST)
        pooled[j] = lax.dot_general(ph, pmat, (((1,), (0,)), ((), ())),
                                    preferred_element_type=jnp.float32,
                                    precision=lax.Precision.HIGHEST)
    for b in range(_NBINS):
        if b == _NBINS - 1:
            o_ref[0, b] = pooled[b]
        else:
            o_ref[0, b] = pooled[b] - pooled[b + 1]


@jax.jit
def kernel(x):
    n = x.shape[0]
    x2 = x.reshape(n, _H, _W)
    out = pl.pallas_call(
        _hog_body,
        grid=(n,),
        in_specs=[pl.BlockSpec((1, _H, _W), lambda i: (i, 0, 0))],
        out_specs=pl.BlockSpec((1, _NBINS, _H // _POOL, _W // _POOL),
                               lambda i: (i, 0, 0, 0)),
        out_shape=jax.ShapeDtypeStruct((n, _NBINS, _H // _POOL, _W // _POOL),
                                       jnp.float32),
    )(x2)
    return out


# speedup vs baseline: 2.3437x; 2.3437x over previous
import math
import functools

import jax
import jax.numpy as jnp
from jax import lax
from jax.experimental import pallas as pl

_NBINS = 10
_POOL = 8
_H = 512
_W = 512


def _hog_body(x_ref, o_ref):
    img = x_ref[0].astype(jnp.bfloat16).astype(jnp.float32)
    zrow = jnp.zeros((1, _W), dtype=jnp.float32)
    zcol = jnp.zeros((_H, 1), dtype=jnp.float32)
    x_up = jnp.concatenate([img[1:, :], zrow], axis=0)
    x_dn = jnp.concatenate([zrow, img[:-1, :]], axis=0)

    t1 = x_dn + 2.0 * img + x_up
    t2 = x_dn - x_up

    t1_l = jnp.concatenate([zcol, t1[:, :-1]], axis=1)
    t1_r = jnp.concatenate([t1[:, 1:], zcol], axis=1)
    t2_l = jnp.concatenate([zcol, t2[:, :-1]], axis=1)
    t2_r = jnp.concatenate([t2[:, 1:], zcol], axis=1)

    gx = t1_l - t1_r
    gy = t2_l + 2.0 * t2 + t2_r

    mag = jnp.sqrt(gx * gx + gy * gy)

    s = jnp.where(gx > 0.0, 1.0, jnp.where(gx < 0.0, -1.0,
                  jnp.where(gy < 0.0, -1.0, 1.0))).astype(jnp.float32)
    u = s * gx
    v = s * gy


    rows = lax.broadcasted_iota(jnp.int32, (_H, _H // _POOL), 0)
    cols = lax.broadcasted_iota(jnp.int32, (_H, _H // _POOL), 1)
    pmat = jnp.where(rows // _POOL == cols, 1.0 / _POOL, 0.0).astype(jnp.float32)

    pooled = [None] * (_NBINS + 1)
    for j in range(_NBINS):
        if j == 0:
            mj = mag
        else:
            cj = (u * jnp.float32(math.cos(j * math.pi / _NBINS))
                  - v * jnp.float32(math.sin(j * math.pi / _NBINS))) >= 0.0
            mj = jnp.where(cj, mag, 0.0)
        ph = lax.dot_general(pmat, mj, (((0,), (0,)), ((), ())),
                             preferred_element_type=jnp.float32)
        pooled[j] = lax.dot_general(ph, pmat, (((1,), (0,)), ((), ())),
                                    preferred_element_type=jnp.float32)
    for b in range(_NBINS):
        if b == _NBINS - 1:
            o_ref[0, b] = pooled[b]
        else:
            o_ref[0, b] = pooled[b] - pooled[b + 1]


@jax.jit
def kernel(x):
    n = x.shape[0]
    x2 = x.reshape(n, _H, _W)
    out = pl.pallas_call(
        _hog_body,
        grid=(n,),
        in_specs=[pl.BlockSpec((1, _H, _W), lambda i: (i, 0, 0))],
        out_specs=pl.BlockSpec((1, _NBINS, _H // _POOL, _W // _POOL),
                               lambda i: (i, 0, 0, 0)),
        out_shape=jax.ShapeDtypeStruct((n, _NBINS, _H // _POOL, _W // _POOL),
                                       jnp.float32),
    )(x2)
    return out
